# baseline (speedup 1.0000x reference)
import functools

import jax
import jax.numpy as jnp
from jax import lax
from jax.experimental import pallas as pl
from jax.experimental.pallas import tpu as pltpu
from jax.experimental.pallas import tpu_sc as plsc

_N_FREQ_MASKS = 2
_FREQ_MASK_SIZE = 27.0
_N_TIME_MASKS = 2
_TIME_MASK_PCT = 0.05

_BBLK = 4
_NC = 2
_NS = 16
_CH = 16


def _tc_body(frand_ref, x_ref, o_ref):
    _, T, F = x_ref.shape
    f_idx = jax.lax.broadcasted_iota(jnp.int32, (1, F), 1)
    fkeep = jnp.ones((1, F), jnp.float32)
    for i in range(_N_FREQ_MASKS):
        value = frand_ref[i, 0] * _FREQ_MASK_SIZE
        min_v = frand_ref[i, 1] * (jnp.float32(F) - value)
        start = jnp.floor(min_v)
        end = start + jnp.floor(value)
        band = (f_idx >= start.astype(jnp.int32)) & (f_idx < end.astype(jnp.int32))
        fkeep = jnp.where(band, jnp.float32(0.0), fkeep)
    for k in range(_BBLK):
        o_ref[k] = x_ref[k] * fkeep


def _sc_body(n_tasks, F, y_hbm, base_hbm, nrow_hbm, tok_hbm, base_v, nrow_v, zer_v):
    wid = lax.axis_index("s") * _NC + lax.axis_index("c")
    tasks_per_w = n_tasks // (_NC * _NS)

    pltpu.sync_copy(base_hbm, base_v.at[pl.ds(0, n_tasks)])
    pltpu.sync_copy(nrow_hbm, nrow_v.at[pl.ds(0, n_tasks)])
    for i in range(_CH * F // 16):
        zer_v[pl.ds(i * 16, 16)] = jnp.zeros((16,), jnp.float32)

    for kk in range(tasks_per_w):
        task = wid * tasks_per_w + kk
        base = pl.multiple_of(base_v[pl.ds(task, 16)][0], F)
        nrow = nrow_v[pl.ds(task, 16)][0]
        nf = nrow // _CH
        rem = nrow - nf * _CH

        def chunk_body(i, _, base=base):
            pltpu.sync_copy(zer_v, y_hbm.at[pl.ds(base + i * (_CH * F), _CH * F)])
            return 0

        lax.fori_loop(0, nf, chunk_body, 0)

        def row_body(i, _, base=base, nf=nf):
            pltpu.sync_copy(
                zer_v.at[pl.ds(0, F)],
                y_hbm.at[pl.ds(base + (nf * _CH + i) * F, F)],
            )
            return 0

        lax.fori_loop(0, rem, row_body, 0)

    @pl.when(wid == 0)
    def _():
        pltpu.sync_copy(y_hbm.at[pl.ds(0, 128)], tok_hbm)


def kernel(x, x_len, freq_rand, time_rand):
    B, T, F = x.shape
    y = pl.pallas_call(
        _tc_body,
        grid=(B // _BBLK,),
        in_specs=[
            pl.BlockSpec(memory_space=pltpu.SMEM),
            pl.BlockSpec((_BBLK, T, F), lambda b: (b, 0, 0)),
        ],
        out_specs=pl.BlockSpec((_BBLK, T, F), lambda b: (b, 0, 0)),
        out_shape=jax.ShapeDtypeStruct((B, T, F), x.dtype),
        compiler_params=pltpu.CompilerParams(
            dimension_semantics=("parallel",),
        ),
    )(freq_rand, x)

    xlen_f = x_len.astype(jnp.float32)
    param = jnp.floor(_TIME_MASK_PCT * xlen_f)
    value = time_rand[:, :, 0] * param[:, None]
    min_v = time_rand[:, :, 1] * (xlen_f[:, None] - value)
    istart = jnp.floor(min_v).astype(jnp.int32)
    nrows = jnp.floor(value).astype(jnp.int32)
    b_idx = jnp.arange(B, dtype=jnp.int32)[:, None]
    base = (b_idx * T + istart) * F
    n_tasks = B * _N_TIME_MASKS

    y1 = y.reshape(B * T * F)
    mesh = plsc.VectorSubcoreMesh(
        core_axis_name="c", subcore_axis_name="s", num_cores=_NC, num_subcores=_NS
    )
    sc_fix = functools.partial(
        pl.kernel,
        out_type=jax.ShapeDtypeStruct((128,), jnp.float32),
        mesh=mesh,
        scratch_types=[
            pltpu.VMEM((n_tasks + 16,), jnp.int32),
            pltpu.VMEM((n_tasks + 16,), jnp.int32),
            pltpu.VMEM((_CH * F,), jnp.float32),
        ],
    )(functools.partial(_sc_body, n_tasks, F))
    tok = sc_fix(y1, base.reshape(-1), nrows.reshape(-1))
    out = lax.dynamic_update_slice(y1, tok, (jnp.int32(0),))
    return out.reshape(B, T, F)
